# Initial kernel scaffold; baseline (speedup 1.0000x reference)
import jax
import jax.numpy as jnp
from jax.experimental import pallas as pl


def kernel(idx, token_embedding_table):
    raise NotImplementedError("write your pallas kernel here")


# speedup vs baseline: 1.4595x; 1.4595x over previous
import functools

import jax
import jax.numpy as jnp
from jax import lax
from jax.experimental import pallas as pl
from jax.experimental.pallas import tpu as pltpu
from jax.experimental.pallas import tpu_sc as plsc

NC, NS = 2, 16
NW = NC * NS
CH = 1024


@jax.jit
def _run(flat_idx, table):
    n = flat_idx.shape[0]
    d = table.shape[1]
    b_per_w = n // NW
    n_ch = b_per_w // CH

    mesh = plsc.VectorSubcoreMesh(
        core_axis_name="c", subcore_axis_name="s",
        num_cores=NC, num_subcores=NS,
    )

    @functools.partial(
        pl.kernel,
        mesh=mesh,
        out_type=jax.ShapeDtypeStruct((n, d), jnp.float32),
        scratch_types=[
            pltpu.VMEM((CH,), jnp.int32),
            pltpu.VMEM((CH, d), jnp.float32),
            pltpu.SemaphoreType.DMA,
        ],
        compiler_params=pltpu.CompilerParams(use_tc_tiling_on_sc=False),
    )
    def k(idx_hbm, table_hbm, out_hbm, idx_v, rows_v, sem):
        wid = lax.axis_index("s") * NC + lax.axis_index("c")
        base = wid * b_per_w

        @pl.loop(0, n_ch)
        def _(c):
            off = base + c * CH
            pltpu.sync_copy(idx_hbm.at[pl.ds(off, CH)], idx_v)
            pltpu.async_copy(table_hbm.at[idx_v], rows_v, sem).wait()
            pltpu.sync_copy(rows_v, out_hbm.at[pl.ds(off, CH)])

    return k(flat_idx, table)


def kernel(idx, token_embedding_table):
    b, t = idx.shape
    d = token_embedding_table.shape[1]
    flat_idx = idx.reshape(-1).astype(jnp.int32)
    out = _run(flat_idx, token_embedding_table)
    return out.reshape(b, t, d)


# speedup vs baseline: 1.4932x; 1.0231x over previous
import functools

import jax
import jax.numpy as jnp
from jax import lax
from jax.experimental import pallas as pl
from jax.experimental.pallas import tpu as pltpu
from jax.experimental.pallas import tpu_sc as plsc

NC, NS = 2, 16
NW = NC * NS
CH = 640
NBUF = 4


@jax.jit
def _run(flat_idx, table):
    n = flat_idx.shape[0]
    d = table.shape[1]
    b_per_w = n // NW
    n_ch = b_per_w // CH
    n_rounds = n_ch // NBUF

    mesh = plsc.VectorSubcoreMesh(
        core_axis_name="c", subcore_axis_name="s",
        num_cores=NC, num_subcores=NS,
    )

    @functools.partial(
        pl.kernel,
        mesh=mesh,
        out_type=jax.ShapeDtypeStruct((n, d), jnp.float32),
        scratch_types=[
            pltpu.VMEM((b_per_w,), jnp.int32),
            [pltpu.VMEM((CH, d), jnp.float32) for _ in range(NBUF)],
            [pltpu.SemaphoreType.DMA for _ in range(NBUF)],
            [pltpu.SemaphoreType.DMA for _ in range(NBUF)],
        ],
        compiler_params=pltpu.CompilerParams(use_tc_tiling_on_sc=False),
    )
    def k(idx_hbm, table_hbm, out_hbm, idx_all, rows, ssem, gsem):
        wid = lax.axis_index("s") * NC + lax.axis_index("c")
        base = wid * b_per_w
        pltpu.sync_copy(idx_hbm.at[pl.ds(base, b_per_w)], idx_all)

        @pl.loop(0, n_rounds)
        def _(r):
            c0 = r * NBUF
            gathers = []
            for b in range(NBUF):
                @pl.when(r > 0)
                def _():
                    prev = base + (c0 + b - NBUF) * CH
                    pltpu.make_async_copy(
                        rows[b], out_hbm.at[pl.ds(prev, CH)], ssem[b]
                    ).wait()

                loc = (c0 + b) * CH
                gathers.append(pltpu.async_copy(
                    table_hbm.at[idx_all.at[pl.ds(loc, CH)]], rows[b], gsem[b]
                ))
            for b in range(NBUF):
                gathers[b].wait()
                pltpu.async_copy(
                    rows[b], out_hbm.at[pl.ds(base + (c0 + b) * CH, CH)],
                    ssem[b],
                )

        for b in range(NBUF):
            last = base + (n_ch - NBUF + b) * CH
            pltpu.make_async_copy(
                rows[b], out_hbm.at[pl.ds(last, CH)], ssem[b]
            ).wait()

    return k(flat_idx, table)


def kernel(idx, token_embedding_table):
    b, t = idx.shape
    d = token_embedding_table.shape[1]
    flat_idx = idx.reshape(-1).astype(jnp.int32)
    out = _run(flat_idx, token_embedding_table)
    return out.reshape(b, t, d)
